# Initial kernel scaffold; baseline (speedup 1.0000x reference)
import jax
import jax.numpy as jnp
from jax.experimental import pallas as pl


def kernel(x, affinity_matrix, num_spixels, ln_w, ln_b, Wq, Wk, Wv):
    raise NotImplementedError("write your pallas kernel here")


# speedup vs baseline: 4.7661x; 4.7661x over previous
import functools

import jax
import jax.numpy as jnp
from jax import lax
from jax.experimental import pallas as pl
from jax.experimental.pallas import tpu as pltpu
from jax.experimental.pallas import tpu_sc as plsc

_HEADS = 6
_TOPK = 32
_NEG = -1e30


def _proj_body(x_ref, w_ref, lnw_ref, lnb_ref, out_ref):
    xb = x_ref[0]
    mu = jnp.mean(xb, axis=1, keepdims=True)
    xc = xb - mu
    var = jnp.mean(xc * xc, axis=1, keepdims=True)
    xn = xc * lax.rsqrt(var + 1e-6) * lnw_ref[0] + lnb_ref[0]
    qkv = lax.dot_general(xn, w_ref[...], (((1,), (1,)), ((), ())),
                          preferred_element_type=jnp.float32)
    pad = out_ref.shape[2] - qkv.shape[1]
    out_ref[0] = jnp.concatenate(
        [qkv, jnp.zeros((qkv.shape[0], pad), jnp.float32)], axis=1)


def _proj(xT, w_all, ln_w, ln_b, tile=512):
    B, HW, C = xT.shape
    O = w_all.shape[0]
    OP = O + (-O) % 128
    return pl.pallas_call(
        _proj_body,
        grid=(B, HW // tile),
        in_specs=[
            pl.BlockSpec((1, tile, C), lambda b, i: (b, i, 0)),
            pl.BlockSpec((O, C), lambda b, i: (0, 0)),
            pl.BlockSpec((1, C), lambda b, i: (0, 0)),
            pl.BlockSpec((1, C), lambda b, i: (0, 0)),
        ],
        out_specs=pl.BlockSpec((1, tile, OP), lambda b, i: (b, i, 0)),
        out_shape=jax.ShapeDtypeStruct((B, HW, OP), jnp.float32),
    )(xT, w_all, ln_w[None], ln_b[None])


def _topk_body(aff_ref, sims_ref, idx_ref, scr):
    R, HW = scr.shape
    scr[...] = aff_ref[0]
    iota = lax.broadcasted_iota(jnp.int32, (R, HW), 1)
    sims, idxs = [], []
    for _ in range(_TOPK):
        cur = scr[...]
        m = jnp.max(cur, axis=1, keepdims=True)
        cand = jnp.where(cur == m, iota, jnp.int32(HW))
        am = jnp.min(cand, axis=1, keepdims=True)
        sims.append(m)
        idxs.append(am)
        scr[...] = jnp.where(iota == am, -jnp.inf, cur)
    sims_ref[0] = jnp.concatenate(sims, axis=1)
    idx_ref[0] = jnp.concatenate(idxs, axis=1)


def _topk(aff, rows=128):
    B, K, HW = aff.shape
    return pl.pallas_call(
        _topk_body,
        grid=(B, K // rows),
        in_specs=[pl.BlockSpec((1, rows, HW), lambda b, i: (b, i, 0))],
        out_specs=[
            pl.BlockSpec((1, rows, _TOPK), lambda b, i: (b, i, 0)),
            pl.BlockSpec((1, rows, _TOPK), lambda b, i: (b, i, 0)),
        ],
        out_shape=[
            jax.ShapeDtypeStruct((B, K, _TOPK), jnp.float32),
            jax.ShapeDtypeStruct((B, K, _TOPK), jnp.int32),
        ],
        scratch_shapes=[pltpu.VMEM((rows, HW), jnp.float32)],
    )(aff)


def _sc_gather(table, gidx, chunk=128):
    NROWS = gidx.shape[0]
    D = table.shape[1]
    info = plsc.get_sparse_core_info()
    NW = info.num_cores * info.num_subcores
    rows_w = NROWS // NW
    nch = rows_w // chunk
    nc = info.num_cores

    @functools.partial(
        pl.kernel,
        out_type=jax.ShapeDtypeStruct((NROWS, D), jnp.float32),
        mesh=plsc.VectorSubcoreMesh(core_axis_name="c", subcore_axis_name="s"),
        scratch_types=[
            pltpu.VMEM((chunk,), jnp.int32),
            pltpu.VMEM((chunk, D), jnp.float32),
            pltpu.SemaphoreType.DMA,
        ],
    )
    def _gather(table_hbm, idx_hbm, out_hbm, idx_v, rows_v, sem):
        c = lax.axis_index("c")
        s = lax.axis_index("s")
        base = (s * nc + c) * rows_w

        def body(j, carry):
            off = base + j * chunk
            pltpu.sync_copy(idx_hbm.at[pl.ds(off, chunk)], idx_v)
            pltpu.async_copy(table_hbm.at[idx_v], rows_v, sem).wait()
            pltpu.sync_copy(rows_v, out_hbm.at[pl.ds(off, chunk)])
            return carry

        lax.fori_loop(0, nch, body, 0)

    return _gather(table, gidx)


def _attn_body(g_ref, w_ref, src_ref, *, C):
    SP = g_ref.shape[0]
    hd = C // _HEADS
    scale = hd ** (-0.5)
    bi = lax.broadcasted_iota(jnp.int32, (C, C), 0) // hd
    bj = lax.broadcasted_iota(jnp.int32, (C, C), 1) // hd
    maskM = jnp.where(bi == bj, 0.0, _NEG).astype(jnp.float32)
    for p in range(SP):
        G = g_ref[p]
        w = w_ref[p]
        Q = G[:, 0:C]
        Km = G[:, C:2 * C]
        V = w * G[:, 2 * C:3 * C]
        Qr = jnp.concatenate([Q[:, h * hd:(h + 1) * hd] for h in range(_HEADS)], axis=0)
        Kr = jnp.concatenate([Km[:, h * hd:(h + 1) * hd] for h in range(_HEADS)], axis=0)
        Vr = jnp.concatenate([V[:, h * hd:(h + 1) * hd] for h in range(_HEADS)], axis=0)
        S = lax.dot_general(Qr, Kr, (((1,), (1,)), ((), ())),
                            preferred_element_type=jnp.float32)
        S = S * scale + maskM
        m = jnp.max(S, axis=1, keepdims=True)
        P = jnp.exp(S - m)
        P = P / jnp.sum(P, axis=1, keepdims=True)
        O = lax.dot_general(P, Vr, (((1,), (0,)), ((), ())),
                            preferred_element_type=jnp.float32)
        parts = [w * O[h * _TOPK:(h + 1) * _TOPK, :] for h in range(_HEADS)]
        pad = src_ref.shape[2] - C
        if pad:
            parts.append(jnp.zeros((_TOPK, pad), jnp.float32))
        src_ref[p] = jnp.concatenate(parts, axis=1)


def _attn(g3, simsC, C, sp=8):
    BK, T, D3P = g3.shape
    CP = C + (-C) % 128
    return pl.pallas_call(
        functools.partial(_attn_body, C=C),
        grid=(BK // sp,),
        in_specs=[
            pl.BlockSpec((sp, T, D3P), lambda i: (i, 0, 0)),
            pl.BlockSpec((sp, T, 1), lambda i: (i, 0, 0)),
        ],
        out_specs=pl.BlockSpec((sp, T, CP), lambda i: (i, 0, 0)),
        out_shape=jax.ShapeDtypeStruct((BK, T, CP), jnp.float32),
    )(g3, simsC)


def _sc_scatter(src, gdst, zeros, B, HW, chunk=128):
    NR, C = src.shape
    rows_b = NR // B
    info = plsc.get_sparse_core_info()
    NW = info.num_cores * info.num_subcores
    nc = info.num_cores
    rows_w = rows_b // NW
    nch = rows_w // chunk
    zrows = HW // info.num_subcores

    @functools.partial(
        pl.kernel,
        out_type=jax.ShapeDtypeStruct((nc * B * HW, C), jnp.float32),
        mesh=plsc.VectorSubcoreMesh(core_axis_name="c", subcore_axis_name="s"),
        scratch_types=[
            pltpu.VMEM((chunk,), jnp.int32),
            pltpu.VMEM((chunk, C), jnp.float32),
            pltpu.SemaphoreType.DMA,
        ],
    )
    def _scatter(src_hbm, idx_hbm, zeros_hbm, out_hbm, idx_v, rows_v, sem):
        c = lax.axis_index("c")
        s = lax.axis_index("s")
        wid = s * nc + c
        for b in range(B):
            start = (c * B + b) * HW + s * zrows
            pltpu.sync_copy(zeros_hbm, out_hbm.at[pl.ds(start, zrows)])
        plsc.subcore_barrier()

        def body(j, carry):
            off = j * chunk
            pltpu.sync_copy(src_hbm.at[pl.ds(off, chunk)], rows_v)
            pltpu.sync_copy(idx_hbm.at[pl.ds(off, chunk)], idx_v)
            pltpu.async_copy(rows_v, out_hbm.at[idx_v], sem, add=True).wait()
            return carry

        for b in range(B):
            base = b * rows_b + wid * rows_w
            lax.fori_loop(base // chunk, base // chunk + nch, body, 0)

    return _scatter(src, gdst, zeros)


def _comb_body(qkv_ref, p_ref, out_ref):
    C = out_ref.shape[2]
    out_ref[0] = (qkv_ref[0][:, 2 * C:3 * C]
                  + p_ref[0, 0][:, :C] + p_ref[1, 0][:, :C])


def _combine(qkv, partials, C, tile=512):
    B, HW, D3P = qkv.shape
    CP = partials.shape[3]
    return pl.pallas_call(
        _comb_body,
        grid=(B, HW // tile),
        in_specs=[
            pl.BlockSpec((1, tile, D3P), lambda b, i: (b, i, 0)),
            pl.BlockSpec((2, 1, tile, CP), lambda b, i: (0, b, i, 0)),
        ],
        out_specs=pl.BlockSpec((1, tile, C), lambda b, i: (b, i, 0)),
        out_shape=jax.ShapeDtypeStruct((B, HW, C), jnp.float32),
    )(qkv, partials)


def kernel(x, affinity_matrix, num_spixels, ln_w, ln_b, Wq, Wk, Wv):
    B, C, H, W = x.shape
    HW = H * W
    K = affinity_matrix.shape[1]

    xT = jnp.transpose(x.reshape(B, C, HW), (0, 2, 1))
    w_all = jnp.concatenate([Wq, Wk, Wv], axis=0)
    qkv = _proj(xT, w_all, ln_w, ln_b)
    D3P = qkv.shape[2]

    sims, idx = _topk(affinity_matrix)
    offs = (jnp.arange(B, dtype=jnp.int32) * HW)[:, None, None]
    gidx = (idx + offs).reshape(B * K * _TOPK)

    gath = _sc_gather(qkv.reshape(B * HW, D3P), gidx)
    src = _attn(gath.reshape(B * K, _TOPK, D3P),
                sims.reshape(B * K, _TOPK, 1), C)
    CP = src.shape[2]

    NR = B * K * _TOPK
    rows_b = K * _TOPK
    rows_w = rows_b // 32
    pos = jnp.arange(NR, dtype=jnp.int32)
    core = (pos % rows_b) // rows_w % 2
    batch = pos // rows_b
    gdst = (core * B + batch) * HW + idx.reshape(NR)

    zeros = jnp.zeros((HW // 16, CP), jnp.float32)
    if True:
        acc = jnp.zeros((B, HW, C), jnp.float32)
        accv = jax.vmap(lambda a, i, s: a.at[i].add(s))(
            acc, idx.reshape(B, K * _TOPK), src.reshape(B, K * _TOPK, CP)[:, :, :C])
        out = qkv[:, :, 2 * C:3 * C] + accv
    else:
        partials = _sc_scatter(src.reshape(NR, CP), gdst, zeros, B, HW)
        out = _combine(qkv, partials.reshape(2, B, HW, CP), C)
    return jnp.transpose(out, (0, 2, 1)).reshape(B, C, H, W)


# speedup vs baseline: 4.8034x; 1.0078x over previous
import functools

import jax
import jax.numpy as jnp
from jax import lax
from jax.experimental import pallas as pl
from jax.experimental.pallas import tpu as pltpu
from jax.experimental.pallas import tpu_sc as plsc

_HEADS = 6
_TOPK = 32
_NEG = -1e30


def _proj_body(x_ref, w_ref, lnw_ref, lnb_ref, out_ref):
    xb = x_ref[0]
    mu = jnp.mean(xb, axis=1, keepdims=True)
    xc = xb - mu
    var = jnp.mean(xc * xc, axis=1, keepdims=True)
    xn = xc * lax.rsqrt(var + 1e-6) * lnw_ref[0] + lnb_ref[0]
    qkv = lax.dot_general(xn, w_ref[...], (((1,), (1,)), ((), ())),
                          preferred_element_type=jnp.float32)
    pad = out_ref.shape[2] - qkv.shape[1]
    out_ref[0] = jnp.concatenate(
        [qkv, jnp.zeros((qkv.shape[0], pad), jnp.float32)], axis=1)


def _proj(xT, w_all, ln_w, ln_b, tile=512):
    B, HW, C = xT.shape
    O = w_all.shape[0]
    OP = O + (-O) % 128
    return pl.pallas_call(
        _proj_body,
        grid=(B, HW // tile),
        in_specs=[
            pl.BlockSpec((1, tile, C), lambda b, i: (b, i, 0)),
            pl.BlockSpec((O, C), lambda b, i: (0, 0)),
            pl.BlockSpec((1, C), lambda b, i: (0, 0)),
            pl.BlockSpec((1, C), lambda b, i: (0, 0)),
        ],
        out_specs=pl.BlockSpec((1, tile, OP), lambda b, i: (b, i, 0)),
        out_shape=jax.ShapeDtypeStruct((B, HW, OP), jnp.float32),
    )(xT, w_all, ln_w[None], ln_b[None])


def _topk_body(aff_ref, sims_ref, idx_ref, scr):
    R, HW = scr.shape
    scr[...] = aff_ref[0]
    iota = lax.broadcasted_iota(jnp.int32, (R, HW), 1)
    sims, idxs = [], []
    for _ in range(_TOPK):
        cur = scr[...]
        m = jnp.max(cur, axis=1, keepdims=True)
        cand = jnp.where(cur == m, iota, jnp.int32(HW))
        am = jnp.min(cand, axis=1, keepdims=True)
        sims.append(m)
        idxs.append(am)
        scr[...] = jnp.where(iota == am, -jnp.inf, cur)
    sims_ref[0] = jnp.concatenate(sims, axis=1)
    idx_ref[0] = jnp.concatenate(idxs, axis=1)


def _topk(aff, rows=128):
    B, K, HW = aff.shape
    return pl.pallas_call(
        _topk_body,
        grid=(B, K // rows),
        in_specs=[pl.BlockSpec((1, rows, HW), lambda b, i: (b, i, 0))],
        out_specs=[
            pl.BlockSpec((1, rows, _TOPK), lambda b, i: (b, i, 0)),
            pl.BlockSpec((1, rows, _TOPK), lambda b, i: (b, i, 0)),
        ],
        out_shape=[
            jax.ShapeDtypeStruct((B, K, _TOPK), jnp.float32),
            jax.ShapeDtypeStruct((B, K, _TOPK), jnp.int32),
        ],
        scratch_shapes=[pltpu.VMEM((rows, HW), jnp.float32)],
    )(aff)


def _sc_gather(table, gidx, chunk=128):
    NROWS = gidx.shape[0]
    D = table.shape[1]
    info = plsc.get_sparse_core_info()
    NW = info.num_cores * info.num_subcores
    rows_w = NROWS // NW
    nch = rows_w // chunk
    nc = info.num_cores

    @functools.partial(
        pl.kernel,
        out_type=jax.ShapeDtypeStruct((NROWS, D), jnp.float32),
        mesh=plsc.VectorSubcoreMesh(core_axis_name="c", subcore_axis_name="s"),
        scratch_types=[
            pltpu.VMEM((chunk,), jnp.int32),
            pltpu.VMEM((chunk, D), jnp.float32),
            pltpu.SemaphoreType.DMA,
        ],
    )
    def _gather(table_hbm, idx_hbm, out_hbm, idx_v, rows_v, sem):
        c = lax.axis_index("c")
        s = lax.axis_index("s")
        base = (s * nc + c) * rows_w

        def body(j, carry):
            off = base + j * chunk
            pltpu.sync_copy(idx_hbm.at[pl.ds(off, chunk)], idx_v)
            pltpu.async_copy(table_hbm.at[idx_v], rows_v, sem).wait()
            pltpu.sync_copy(rows_v, out_hbm.at[pl.ds(off, chunk)])
            return carry

        lax.fori_loop(0, nch, body, 0)

    return _gather(table, gidx)


def _attn_body(g_ref, w_ref, src_ref, *, C):
    SP = g_ref.shape[0]
    hd = C // _HEADS
    scale = hd ** (-0.5)
    bi = lax.broadcasted_iota(jnp.int32, (C, C), 0) // hd
    bj = lax.broadcasted_iota(jnp.int32, (C, C), 1) // hd
    maskM = jnp.where(bi == bj, 0.0, _NEG).astype(jnp.float32)
    for p in range(SP):
        G = g_ref[p]
        w = w_ref[p]
        Q = G[:, 0:C]
        Km = G[:, C:2 * C]
        V = w * G[:, 2 * C:3 * C]
        Qr = jnp.concatenate([Q[:, h * hd:(h + 1) * hd] for h in range(_HEADS)], axis=0)
        Kr = jnp.concatenate([Km[:, h * hd:(h + 1) * hd] for h in range(_HEADS)], axis=0)
        Vr = jnp.concatenate([V[:, h * hd:(h + 1) * hd] for h in range(_HEADS)], axis=0)
        S = lax.dot_general(Qr, Kr, (((1,), (1,)), ((), ())),
                            preferred_element_type=jnp.float32)
        S = S * scale + maskM
        m = jnp.max(S, axis=1, keepdims=True)
        P = jnp.exp(S - m)
        P = P / jnp.sum(P, axis=1, keepdims=True)
        O = lax.dot_general(P, Vr, (((1,), (0,)), ((), ())),
                            preferred_element_type=jnp.float32)
        parts = [w * O[h * _TOPK:(h + 1) * _TOPK, :] for h in range(_HEADS)]
        src_ref[p] = jnp.concatenate(parts, axis=1).astype(jnp.bfloat16)


def _attn(g3, simsC, C, sp=8):
    BK, T, D3P = g3.shape
    return pl.pallas_call(
        functools.partial(_attn_body, C=C),
        grid=(BK // sp,),
        in_specs=[
            pl.BlockSpec((sp, T, D3P), lambda i: (i, 0, 0)),
            pl.BlockSpec((sp, T, 1), lambda i: (i, 0, 0)),
        ],
        out_specs=pl.BlockSpec((sp, T, C), lambda i: (i, 0, 0)),
        out_shape=jax.ShapeDtypeStruct((BK, T, C), jnp.bfloat16),
    )(g3, simsC)


def _scatter_mm_body(idx_ref, src_ref, qkv_ref, out_ref, *, nr, tile):
    p = pl.program_id(1)
    r = pl.program_id(2)
    rchunk = src_ref.shape[1]
    C = out_ref.shape[2]
    iota0 = lax.broadcasted_iota(jnp.int32, (tile, rchunk), 0) + p * tile
    oh = (iota0 == idx_ref[0]).astype(jnp.bfloat16)
    acc = lax.dot_general(oh, src_ref[0], (((1,), (0,)), ((), ())),
                          preferred_element_type=jnp.float32)

    @pl.when(r == 0)
    def _():
        out_ref[0] = acc

    @pl.when(r > 0)
    def _():
        out_ref[0] += acc

    @pl.when(r == nr - 1)
    def _():
        out_ref[0] += qkv_ref[0][:, 2 * C:3 * C]


def _scatter_mm(idx, src, qkv, tile=512, rchunk=2048):
    B, NRB, C = src.shape
    HW = qkv.shape[1]
    D3P = qkv.shape[2]
    nr = NRB // rchunk
    idx3 = idx.reshape(B * nr, 1, rchunk)
    return pl.pallas_call(
        functools.partial(_scatter_mm_body, nr=nr, tile=tile),
        grid=(B, HW // tile, nr),
        in_specs=[
            pl.BlockSpec((1, 1, rchunk), lambda b, p, r: (b * nr + r, 0, 0)),
            pl.BlockSpec((1, rchunk, C), lambda b, p, r: (b, r, 0)),
            pl.BlockSpec((1, tile, D3P), lambda b, p, r: (b, p, 0)),
        ],
        out_specs=pl.BlockSpec((1, tile, C), lambda b, p, r: (b, p, 0)),
        out_shape=jax.ShapeDtypeStruct((B, HW, C), jnp.float32),
    )(idx3, src, qkv)


def kernel(x, affinity_matrix, num_spixels, ln_w, ln_b, Wq, Wk, Wv):
    B, C, H, W = x.shape
    HW = H * W
    K = affinity_matrix.shape[1]

    xT = jnp.transpose(x.reshape(B, C, HW), (0, 2, 1))
    w_all = jnp.concatenate([Wq, Wk, Wv], axis=0)
    qkv = _proj(xT, w_all, ln_w, ln_b)
    D3P = qkv.shape[2]

    sims, idx = _topk(affinity_matrix)
    offs = (jnp.arange(B, dtype=jnp.int32) * HW)[:, None, None]
    gidx = (idx + offs).reshape(B * K * _TOPK)

    gath = _sc_gather(qkv.reshape(B * HW, D3P), gidx)
    src = _attn(gath.reshape(B * K, _TOPK, D3P),
                sims.reshape(B * K, _TOPK, 1), C)
    out = _scatter_mm(idx.reshape(B, K * _TOPK),
                      src.reshape(B, K * _TOPK, C), qkv)
    return jnp.transpose(out, (0, 2, 1)).reshape(B, C, H, W)
